# Initial kernel scaffold; baseline (speedup 1.0000x reference)
import jax
import jax.numpy as jnp
from jax.experimental import pallas as pl


def kernel(z, embed_weight):
    raise NotImplementedError("write your pallas kernel here")


# speedup vs baseline: 1.2234x; 1.2234x over previous
import functools

import jax
import jax.numpy as jnp
from jax import lax
from jax.experimental import pallas as pl
from jax.experimental.pallas import tpu as pltpu
from jax.experimental.pallas import tpu_sc as plsc

N_EMBED = 8192
EMBED_DIM = 32
BETA = 0.25
TOK = 16 * 1024
BLK = 512


def _dist_argmin_body(z_ref, e_ref, idx_ref, dsum_ref):
    z = z_ref[...]
    e = e_ref[...]
    zn = jnp.sum(z * z, axis=1, keepdims=True)
    en = jnp.sum(e * e, axis=1)[None, :]
    dot = lax.dot_general(z, e, (((1,), (1,)), ((), ())),
                          preferred_element_type=jnp.float32)
    d = zn + en - 2.0 * dot
    minv = jnp.min(d, axis=1, keepdims=True)
    ridx = lax.broadcasted_iota(jnp.int32, d.shape, 1)
    idx = jnp.min(jnp.where(d == minv, ridx, N_EMBED), axis=1)
    idx_ref[0, 0, :] = idx
    dsum_ref[0, 0, :] = jnp.full((128,), jnp.sum(minv), dtype=jnp.float32)


def _dist_argmin(zf, embed_weight):
    grid = TOK // BLK
    return pl.pallas_call(
        _dist_argmin_body,
        grid=(grid,),
        in_specs=[
            pl.BlockSpec((BLK, EMBED_DIM), lambda i: (i, 0)),
            pl.BlockSpec((N_EMBED, EMBED_DIM), lambda i: (0, 0)),
        ],
        out_specs=[
            pl.BlockSpec((1, 1, BLK), lambda i: (i, 0, 0)),
            pl.BlockSpec((1, 1, 128), lambda i: (i, 0, 0)),
        ],
        out_shape=[
            jax.ShapeDtypeStruct((grid, 1, BLK), jnp.int32),
            jax.ShapeDtypeStruct((grid, 1, 128), jnp.float32),
        ],
    )(zf, embed_weight)


LANE = 128


def _make_sc_gather():
    info = plsc.get_sparse_core_info()
    nw = info.num_cores * info.num_subcores
    bpw = TOK // nw
    nch = bpw // LANE
    mesh = plsc.VectorSubcoreMesh(core_axis_name="c", subcore_axis_name="s")

    @functools.partial(
        pl.kernel, mesh=mesh,
        out_type=jax.ShapeDtypeStruct((TOK, LANE), jnp.float32),
        scratch_types=[
            [pltpu.VMEM((LANE,), jnp.int32) for _ in range(nch)],
            [pltpu.VMEM((LANE, LANE), jnp.float32) for _ in range(nch)],
            pltpu.SemaphoreType.DMA,
        ],
    )
    def gather(table_hbm, idx_hbm, out_hbm, idx_vs, rows_vs, sem):
        wid = lax.axis_index("s") * info.num_cores + lax.axis_index("c")
        base = wid * bpw
        for j in range(nch):
            pltpu.sync_copy(idx_hbm.at[pl.ds(base + j * LANE, LANE)], idx_vs[j])
        for j in range(nch):
            pltpu.async_copy(table_hbm.at[idx_vs[j]], rows_vs[j], sem)
        for j in range(nch):
            pltpu.make_async_copy(table_hbm.at[idx_vs[j]], rows_vs[j], sem).wait()
        for j in range(nch):
            pltpu.sync_copy(rows_vs[j], out_hbm.at[pl.ds(base + j * LANE, LANE)])

    return gather


def kernel(z, embed_weight):
    zf = z.reshape(-1, EMBED_DIM)
    idx3, dsum3 = _dist_argmin(zf, embed_weight)
    idx = idx3.reshape(TOK)
    table_pad = jnp.pad(embed_weight, ((0, 0), (0, LANE - EMBED_DIM)))
    z_q = _make_sc_gather()(table_pad, idx)[:, :EMBED_DIM].reshape(z.shape)
    m = jnp.sum(dsum3[:, 0, 0]) / (TOK * EMBED_DIM)
    loss = m + BETA * m
    z_q_st = z + (z_q - z)
    return (z_q_st, loss, idx.reshape(z.shape[:-1]))


# speedup vs baseline: 1.3383x; 1.0939x over previous
import functools

import jax
import jax.numpy as jnp
from jax import lax
from jax.experimental import pallas as pl
from jax.experimental.pallas import tpu as pltpu
from jax.experimental.pallas import tpu_sc as plsc

N_EMBED = 8192
EMBED_DIM = 32
BETA = 0.25
TOK = 16 * 1024
BLK = 512


def _dist_argmin_body(z_ref, e_ref, idx_ref, dsum_ref):
    z = z_ref[...]
    e = e_ref[...]
    zn = jnp.sum(z * z, axis=1, keepdims=True)
    en = jnp.sum(e * e, axis=1)[None, :]
    dot = lax.dot_general(z, e, (((1,), (1,)), ((), ())),
                          preferred_element_type=jnp.float32)
    d = zn + en - 2.0 * dot
    idx = jnp.argmin(d, axis=1).astype(jnp.int32)
    minv = jnp.min(d, axis=1)
    idx_ref[0, 0, :] = idx
    dsum_ref[0, 0, :] = jnp.full((128,), jnp.sum(minv), dtype=jnp.float32)


def _dist_argmin(zf, embed_weight):
    grid = TOK // BLK
    return pl.pallas_call(
        _dist_argmin_body,
        grid=(grid,),
        in_specs=[
            pl.BlockSpec((BLK, EMBED_DIM), lambda i: (i, 0)),
            pl.BlockSpec((N_EMBED, EMBED_DIM), lambda i: (0, 0)),
        ],
        out_specs=[
            pl.BlockSpec((1, 1, BLK), lambda i: (i, 0, 0)),
            pl.BlockSpec((1, 1, 128), lambda i: (i, 0, 0)),
        ],
        out_shape=[
            jax.ShapeDtypeStruct((grid, 1, BLK), jnp.int32),
            jax.ShapeDtypeStruct((grid, 1, 128), jnp.float32),
        ],
    )(zf, embed_weight)


LANE = 128


def _make_sc_gather():
    info = plsc.get_sparse_core_info()
    nw = info.num_cores * info.num_subcores
    bpw = TOK // nw
    nch = bpw // LANE
    mesh = plsc.VectorSubcoreMesh(core_axis_name="c", subcore_axis_name="s")

    @functools.partial(
        pl.kernel, mesh=mesh,
        out_type=jax.ShapeDtypeStruct((TOK, LANE), jnp.float32),
        scratch_types=[
            [pltpu.VMEM((LANE,), jnp.int32) for _ in range(nch)],
            [pltpu.VMEM((LANE, LANE), jnp.float32) for _ in range(nch)],
            pltpu.SemaphoreType.DMA,
        ],
    )
    def gather(table_hbm, idx_hbm, out_hbm, idx_vs, rows_vs, sem):
        wid = lax.axis_index("s") * info.num_cores + lax.axis_index("c")
        base = wid * bpw
        for j in range(nch):
            pltpu.sync_copy(idx_hbm.at[pl.ds(base + j * LANE, LANE)], idx_vs[j])
        for j in range(nch):
            pltpu.async_copy(table_hbm.at[idx_vs[j]], rows_vs[j], sem)
        for j in range(nch):
            pltpu.make_async_copy(table_hbm.at[idx_vs[j]], rows_vs[j], sem).wait()
        for j in range(nch):
            pltpu.sync_copy(rows_vs[j], out_hbm.at[pl.ds(base + j * LANE, LANE)])

    return gather


def kernel(z, embed_weight):
    zf = z.reshape(-1, EMBED_DIM)
    idx3, dsum3 = _dist_argmin(zf, embed_weight)
    idx = idx3.reshape(TOK)
    table_pad = jnp.pad(embed_weight, ((0, 0), (0, LANE - EMBED_DIM)))
    z_q = _make_sc_gather()(table_pad, idx)[:, :EMBED_DIM].reshape(z.shape)
    m = jnp.sum(dsum3[:, 0, 0]) / (TOK * EMBED_DIM)
    loss = m + BETA * m
    z_q_st = z + (z_q - z)
    return (z_q_st, loss, idx.reshape(z.shape[:-1]))
